# baseline (speedup 1.0000x reference)
import functools

import jax
import jax.numpy as jnp
from jax import lax
from jax.experimental import pallas as pl
from jax.experimental.pallas import tpu as pltpu
from jax.experimental.pallas import tpu_sc as plsc

_BASE_GSD = 10.0

_NC = 1
_NS = 16
_NW = _NC * _NS
_ROWS_PAD = 256
_B_PER_W = _ROWS_PAD // _NW


def _sc_gather_table(table, gidx):
    mesh = plsc.VectorSubcoreMesh(core_axis_name="c", subcore_axis_name="s",
                                  num_cores=1)

    @functools.partial(
        pl.kernel,
        mesh=mesh,
        out_type=jax.ShapeDtypeStruct((_ROWS_PAD, 256), jnp.float32),
        scratch_types=[
            pltpu.VMEM((_B_PER_W,), jnp.int32),
            pltpu.VMEM((_B_PER_W, 256), jnp.float32),
            pltpu.SemaphoreType.DMA,
        ],
    )
    def sc_kernel(table_hbm, gidx_hbm, out_hbm, idx_v, rows_v, sem):
        wid = lax.axis_index("s") * _NC + lax.axis_index("c")
        base = wid * _B_PER_W
        pltpu.sync_copy(gidx_hbm.at[pl.ds(base, _B_PER_W)], idx_v)
        pltpu.async_copy(table_hbm.at[idx_v], rows_v, sem).wait()
        pltpu.sync_copy(rows_v, out_hbm.at[pl.ds(base, _B_PER_W)])

    return sc_kernel(table, gidx)


def _tc_body(x_ref, a_ref, angh_ref, angw_ref, o_ref):
    hb = angh_ref.shape[0]
    w = angw_ref.shape[0]
    angh = angh_ref[:, 0]
    angw = angw_ref[:, 0]
    eh = jnp.concatenate([jnp.sin(angh), jnp.cos(angh)], axis=-1)
    ew = jnp.concatenate([jnp.sin(angw), jnp.cos(angw)], axis=-1)
    sp = jnp.concatenate(
        [jnp.broadcast_to(eh[:, None, :], (hb, w, 128)),
         jnp.broadcast_to(ew[None, :, :], (hb, w, 128))], axis=-1)

    o_ref[:, :, :, :, 0:768] = x_ref[:, :, :, :, 0:768] + a_ref[...][None, None]
    o_ref[:, :, :, :, 768:1024] = (
        x_ref[:, :, :, :, 768:1024] + sp[:, :, None, None, :])


def kernel(modality_tokens, timestamps, channel_embed, pos_embed, month_table,
           patch_size, input_res):
    b, h, w, t, b_s, d = modality_tokens.shape
    n = d // 4
    half = n // 2
    ts = t * b_s

    n_ch = channel_embed.shape[0]
    n_pos = pos_embed.shape[0]
    table = jnp.concatenate(
        [channel_embed, pos_embed, month_table], axis=0).astype(jnp.float32)
    t_r = jnp.repeat(jnp.arange(t, dtype=jnp.int32), b_s)
    s_r = jnp.tile(jnp.arange(b_s, dtype=jnp.int32), t)
    mon = timestamps.astype(jnp.int32).T
    q0 = jnp.broadcast_to(s_r[:, None, None], (ts, b, 1))
    q1 = jnp.broadcast_to((n_ch + t_r)[:, None, None], (ts, b, 1))
    q2 = (n_ch + n_pos
          + jnp.broadcast_to(jnp.repeat(mon, b_s, axis=0)[:, :, None],
                             (ts, b, 1)))
    gidx = jnp.concatenate([q0, q1, q2], axis=-1).reshape(-1)
    gidx = jnp.concatenate(
        [gidx, jnp.zeros((_ROWS_PAD - gidx.shape[0],), jnp.int32)])

    a_flat = _sc_gather_table(table, gidx)
    a = a_flat[: ts * b * 3].reshape(ts, b, 3 * n)

    gsd_ratio = (input_res * patch_size) / _BASE_GSD
    omega = 1.0 / (10000.0 ** (jnp.arange(half // 2, dtype=jnp.float32)
                               / (half // 2)))
    ang_h = ((jnp.arange(h, dtype=jnp.float32) * gsd_ratio)[:, None]
             * omega[None, :]).reshape(h, 1, half // 2)
    ang_w = ((jnp.arange(w, dtype=jnp.float32) * gsd_ratio)[:, None]
             * omega[None, :]).reshape(w, 1, half // 2)

    hb = 2
    xt = modality_tokens.transpose(1, 2, 3, 4, 0, 5).reshape(h, w, ts, b, d)
    h1 = 8
    def tc_call(x_part, angh_part, hpart):
        return pl.pallas_call(
            _tc_body,
            grid=(hpart // hb,),
            in_specs=[
                pl.BlockSpec((hb, w, ts, b, d), lambda i: (i, 0, 0, 0, 0)),
                pl.BlockSpec((ts, b, 3 * n), lambda i: (0, 0, 0)),
                pl.BlockSpec((hb, 1, half // 2), lambda i: (i, 0, 0)),
                pl.BlockSpec((w, 1, half // 2), lambda i: (0, 0, 0)),
            ],
            out_specs=pl.BlockSpec((hb, w, ts, b, d),
                                   lambda i: (i, 0, 0, 0, 0)),
            out_shape=jax.ShapeDtypeStruct((hpart, w, ts, b, d), jnp.float32),
        )(x_part, a, angh_part, ang_w)
    out1 = tc_call(xt[:h1], ang_h[:h1], h1)
    out2 = tc_call(xt[h1:], ang_h[h1:], h - h1)
    out = jnp.concatenate([out1, out2], axis=0)
    return (out.reshape(h, w, t, b_s, b, d)
            .transpose(4, 0, 1, 2, 3, 5))


# speedup vs baseline: 2.2475x; 2.2475x over previous
import functools

import jax
import jax.numpy as jnp
from jax import lax
from jax.experimental import pallas as pl
from jax.experimental.pallas import tpu as pltpu
from jax.experimental.pallas import tpu_sc as plsc

_BASE_GSD = 10.0

_NC = 1
_NS = 16
_NW = _NC * _NS
_ROWS_PAD = 256
_B_PER_W = _ROWS_PAD // _NW


def _sc_gather_table(table, gidx):
    mesh = plsc.VectorSubcoreMesh(core_axis_name="c", subcore_axis_name="s",
                                  num_cores=1)

    @functools.partial(
        pl.kernel,
        mesh=mesh,
        out_type=jax.ShapeDtypeStruct((_ROWS_PAD, 256), jnp.float32),
        scratch_types=[
            pltpu.VMEM((_B_PER_W,), jnp.int32),
            pltpu.VMEM((_B_PER_W, 256), jnp.float32),
            pltpu.SemaphoreType.DMA,
        ],
    )
    def sc_kernel(table_hbm, gidx_hbm, out_hbm, idx_v, rows_v, sem):
        wid = lax.axis_index("s") * _NC + lax.axis_index("c")
        base = wid * _B_PER_W
        pltpu.sync_copy(gidx_hbm.at[pl.ds(base, _B_PER_W)], idx_v)
        pltpu.async_copy(table_hbm.at[idx_v], rows_v, sem).wait()
        pltpu.sync_copy(rows_v, out_hbm.at[pl.ds(base, _B_PER_W)])

    return sc_kernel(table, gidx)


def _tc_body(x_ref, a_ref, angh_ref, angw_ref, o_ref):
    hb = angh_ref.shape[0]
    w = angw_ref.shape[0]
    angh = angh_ref[:, 0]
    angw = angw_ref[:, 0]
    eh = jnp.concatenate([jnp.sin(angh), jnp.cos(angh)], axis=-1)
    ew = jnp.concatenate([jnp.sin(angw), jnp.cos(angw)], axis=-1)
    sp = jnp.concatenate(
        [jnp.broadcast_to(eh[:, None, :], (hb, w, 128)),
         jnp.broadcast_to(ew[None, :, :], (hb, w, 128))], axis=-1)

    o_ref[:, :, :, :, 0:768] = x_ref[:, :, :, :, 0:768] + a_ref[...][None, None]
    o_ref[:, :, :, :, 768:1024] = (
        x_ref[:, :, :, :, 768:1024] + sp[:, :, None, None, :])


def kernel(modality_tokens, timestamps, channel_embed, pos_embed, month_table,
           patch_size, input_res):
    b, h, w, t, b_s, d = modality_tokens.shape
    n = d // 4
    half = n // 2
    ts = t * b_s

    n_ch = channel_embed.shape[0]
    n_pos = pos_embed.shape[0]
    table = jnp.concatenate(
        [channel_embed, pos_embed, month_table], axis=0).astype(jnp.float32)
    t_r = jnp.repeat(jnp.arange(t, dtype=jnp.int32), b_s)
    s_r = jnp.tile(jnp.arange(b_s, dtype=jnp.int32), t)
    mon = timestamps.astype(jnp.int32).T
    q0 = jnp.broadcast_to(s_r[:, None, None], (ts, b, 1))
    q1 = jnp.broadcast_to((n_ch + t_r)[:, None, None], (ts, b, 1))
    q2 = (n_ch + n_pos
          + jnp.broadcast_to(jnp.repeat(mon, b_s, axis=0)[:, :, None],
                             (ts, b, 1)))
    gidx = jnp.concatenate([q0, q1, q2], axis=-1).reshape(-1)
    gidx = jnp.concatenate(
        [gidx, jnp.zeros((_ROWS_PAD - gidx.shape[0],), jnp.int32)])

    a_flat = _sc_gather_table(table, gidx)
    a = a_flat[: ts * b * 3].reshape(ts, b, 3 * n)

    gsd_ratio = (input_res * patch_size) / _BASE_GSD
    omega = 1.0 / (10000.0 ** (jnp.arange(half // 2, dtype=jnp.float32)
                               / (half // 2)))
    ang_h = ((jnp.arange(h, dtype=jnp.float32) * gsd_ratio)[:, None]
             * omega[None, :]).reshape(h, 1, half // 2)
    ang_w = ((jnp.arange(w, dtype=jnp.float32) * gsd_ratio)[:, None]
             * omega[None, :]).reshape(w, 1, half // 2)

    hb = 2
    xt = modality_tokens.transpose(1, 2, 3, 4, 0, 5).reshape(h, w, ts, b, d)
    grid = (h // hb,)
    out = pl.pallas_call(
        _tc_body,
        grid=grid,
        in_specs=[
            pl.BlockSpec((hb, w, ts, b, d), lambda i: (i, 0, 0, 0, 0)),
            pl.BlockSpec((ts, b, 3 * n), lambda i: (0, 0, 0)),
            pl.BlockSpec((hb, 1, half // 2), lambda i: (i, 0, 0)),
            pl.BlockSpec((w, 1, half // 2), lambda i: (0, 0, 0)),
        ],
        out_specs=pl.BlockSpec((hb, w, ts, b, d), lambda i: (i, 0, 0, 0, 0)),
        out_shape=jax.ShapeDtypeStruct((h, w, ts, b, d), jnp.float32),
    )(xt, a, ang_h, ang_w)
    return (out.reshape(h, w, t, b_s, b, d)
            .transpose(4, 0, 1, 2, 3, 5))
